# baseline (speedup 1.0000x reference)
import functools
import math

import jax
import jax.numpy as jnp
from jax.experimental import pallas as pl
from jax.experimental.pallas import tpu as pltpu

KEEP_RATIO = 0.5
MIN_TOKENS = 1


def _fused_body(k, H, Q, Tc, aw_ref, hs_ref, out_ref, mask_ref, acc_ref):
    h = pl.program_id(1)

    @pl.when(h < H * Q)
    def _():
        head = h // Q
        col = h % Q
        acc_ref[head, pl.ds(col * Tc, Tc)] = jnp.sum(aw_ref[0, 0], axis=0)

    @pl.when(h == H * Q)
    def _():
        s = jnp.mean(acc_ref[...], axis=0)
        T = s.shape[0]
        s_i = s[:, None]
        s_j = s[None, :]
        i_idx = jax.lax.broadcasted_iota(jnp.int32, (T, T), 0)
        j_idx = jax.lax.broadcasted_iota(jnp.int32, (T, T), 1)
        beats = (s_j > s_i) | ((s_j == s_i) & (j_idx < i_idx))
        rank = jnp.sum(beats.astype(jnp.int32), axis=1)
        pos = jax.lax.broadcasted_iota(jnp.int32, (T,), 0)
        keep = (rank < k) | (pos == 0)
        mask_ref[0, 0, :] = keep.astype(jnp.int32)
        out_ref[0] = hs_ref[0] * keep.astype(out_ref.dtype)[:, None]


@jax.jit
def kernel(hidden_states, attention_weights):
    B, T, D = hidden_states.shape
    _, H, _, _ = attention_weights.shape
    k = min(max(MIN_TOKENS, math.ceil(KEEP_RATIO * T)), T)
    Q = 2 if T >= 256 else 1
    Tc = T // Q

    def _aw_index(b, h):
        last = h >= H * Q
        head = jnp.where(last, H - 1, h // Q)
        col = jnp.where(last, Q - 1, h % Q)
        return (b, head, 0, col)

    pruned, mask_i32 = pl.pallas_call(
        functools.partial(_fused_body, k, H, Q, Tc),
        grid=(B, H * Q + 1),
        in_specs=[
            pl.BlockSpec((1, 1, T, Tc), _aw_index),
            pl.BlockSpec((1, T, D), lambda b, h: (b, 0, 0)),
        ],
        out_specs=[
            pl.BlockSpec((1, T, D), lambda b, h: (b, 0, 0)),
            pl.BlockSpec((1, 1, T), lambda b, h: (b, 0, 0)),
        ],
        out_shape=[
            jax.ShapeDtypeStruct((B, T, D), hidden_states.dtype),
            jax.ShapeDtypeStruct((B, 1, T), jnp.int32),
        ],
        scratch_shapes=[pltpu.VMEM((H, T), jnp.float32)],
        compiler_params=pltpu.CompilerParams(
            dimension_semantics=("arbitrary", "arbitrary"),
        ),
    )(attention_weights, hidden_states)

    return (pruned, mask_i32.reshape(B, T).astype(bool))
